# baseline (speedup 1.0000x reference)
import functools
import jax
import jax.numpy as jnp
from jax import lax
from jax.experimental import pallas as pl
from jax.experimental.pallas import tpu as pltpu, tpu_sc as plsc

DIM = 128
BATCH = 16384
CHUNK = 64
LANES = 16
NW = 32
PER_W = BATCH // NW
NCHUNKS = PER_W // CHUNK

_GDN = lax.GatherDimensionNumbers(
    offset_dims=(), collapsed_slice_dims=(0,), start_index_map=(0,))


def _shuffle(x, idx):
    return lax.gather(x, idx[:, None], dimension_numbers=_GDN,
                      slice_sizes=(1,),
                      mode=lax.GatherScatterMode.PROMISE_IN_BOUNDS)


def _score_body(ent_hbm, rel_hbm, ht_idx, r_idx, out_hbm,
                htix, rix, htbuf, rbuf, qbuf, outv,
                hsem0, rsem0, hsem1, rsem1):
    sems = ((hsem0, rsem0), (hsem1, rsem1))
    nc = 2
    wid = lax.axis_index("s") * nc + lax.axis_index("c")
    base = wid * PER_W

    ih = pltpu.make_async_copy(ht_idx.at[pl.ds(2 * base, 2 * PER_W)],
                               htix, hsem0)
    ir = pltpu.make_async_copy(r_idx.at[pl.ds(base, PER_W)], rix, rsem0)
    ih.start(); ir.start()
    ih.wait(); ir.wait()

    iota = lax.iota(jnp.int32, LANES)
    perms = {s: iota ^ s for s in (8, 4, 2, 1)}

    def copies(c, b):
        hs, rs = sems[b]
        return (pltpu.make_async_copy(
                    ent_hbm.at[htix.at[pl.ds(c * 2 * CHUNK, 2 * CHUNK)]],
                    htbuf.at[b], hs),
                pltpu.make_async_copy(
                    rel_hbm.at[rix.at[pl.ds(c * CHUNK, CHUNK)]],
                    rbuf.at[b], rs))

    def start(c, b):
        for cp in copies(c, b):
            cp.start()

    def wait(c, b):
        for cp in copies(c, b):
            cp.wait()

    def combine(a, bb, s):
        keep = (iota & s) == 0
        return jnp.where(keep, a + _shuffle(a, perms[s]),
                         bb + _shuffle(bb, perms[s]))

    def pair(c, b, i):
        g = lax.shift_right_logical(i, 3)
        ii = jnp.bitwise_and(i, 7)
        j0 = jnp.bitwise_or(
            jnp.bitwise_or(lax.shift_left(jnp.bitwise_and(ii, 1), 2),
                           jnp.bitwise_and(ii, 2)),
            jnp.bitwise_and(lax.shift_right_logical(ii, 2), 1))
        qbase = g * LANES + j0
        accs = []
        for m_off in (0, 8):
            row = qbase + m_off
            acc = jnp.zeros((LANES,), jnp.float32)
            for k in range(DIM // LANES):
                sl_r = pl.ds(k * LANES, LANES)
                sl_i = pl.ds(DIM + k * LANES, LANES)
                hr = htbuf[b, row, sl_r]; hi = htbuf[b, row, sl_i]
                tr = htbuf[b, CHUNK + row, sl_r]
                ti = htbuf[b, CHUNK + row, sl_i]
                p = hr * tr + hi * ti
                q = hr * ti - hi * tr
                rr = rbuf[b, row, sl_r]; ri = rbuf[b, row, sl_i]
                acc = acc + (rr * p + ri * q)
            accs.append(acc)
        qbuf[i, :] = combine(accs[0], accs[1], 8)

    start(0, 0)

    def chunk_pair(cc, carry):
        for b in (0, 1):
            c = 2 * cc + b
            if b == 0:
                start(c + 1, 1)
            else:
                @pl.when(cc < NCHUNKS // 2 - 1)
                def _():
                    start(c + 1, 0)
            wait(c, b)

            def qstep(i2, inner):
                pair(c, b, 2 * i2)
                pair(c, b, 2 * i2 + 1)
                return inner
            lax.fori_loop(0, CHUNK // 4, qstep, jnp.int32(0))

            def gstep(g, inner):
                l2 = [combine(qbuf[8 * g + 2 * i, :],
                              qbuf[8 * g + 2 * i + 1, :], 4)
                      for i in range(4)]
                vec = combine(combine(l2[0], l2[1], 2),
                              combine(l2[2], l2[3], 2), 1)
                outv[pl.ds(c * CHUNK + g * LANES, LANES)] = vec
                return inner
            lax.fori_loop(0, CHUNK // LANES, gstep, jnp.int32(0))
        return carry

    lax.fori_loop(0, NCHUNKS // 2, chunk_pair, jnp.int32(0))

    pltpu.sync_copy(outv, out_hbm.at[pl.ds(base, PER_W)])


def kernel(ent, rel, h, r, t):
    mesh = plsc.VectorSubcoreMesh(core_axis_name="c", subcore_axis_name="s")
    run = functools.partial(
        pl.kernel,
        mesh=mesh,
        out_type=jax.ShapeDtypeStruct((BATCH,), jnp.float32),
        scratch_types=[
            pltpu.VMEM((2 * PER_W,), jnp.int32),
            pltpu.VMEM((PER_W,), jnp.int32),
            pltpu.VMEM((2, 2 * CHUNK, 2 * DIM), jnp.float32),
            pltpu.VMEM((2, CHUNK, 2 * DIM), jnp.float32),
            pltpu.VMEM((CHUNK // 2, LANES), jnp.float32),
            pltpu.VMEM((PER_W,), jnp.float32),
            pltpu.SemaphoreType.DMA,
            pltpu.SemaphoreType.DMA,
            pltpu.SemaphoreType.DMA,
            pltpu.SemaphoreType.DMA,
        ],
    )(_score_body)
    h3 = h.reshape(NW, NCHUNKS, CHUNK)
    t3 = t.reshape(NW, NCHUNKS, CHUNK)
    ht = jnp.concatenate([h3, t3], axis=2).reshape(-1)
    return run(ent, rel, ht, r)


# speedup vs baseline: 1.0159x; 1.0159x over previous
import functools
import jax
import jax.numpy as jnp
from jax import lax
from jax.experimental import pallas as pl
from jax.experimental.pallas import tpu as pltpu, tpu_sc as plsc

DIM = 128
BATCH = 16384
CHUNK = 64
LANES = 16
NW = 32
PER_W = BATCH // NW
NCHUNKS = PER_W // CHUNK

_GDN = lax.GatherDimensionNumbers(
    offset_dims=(), collapsed_slice_dims=(0,), start_index_map=(0,))


def _shuffle(x, idx):
    return lax.gather(x, idx[:, None], dimension_numbers=_GDN,
                      slice_sizes=(1,),
                      mode=lax.GatherScatterMode.PROMISE_IN_BOUNDS)


def _score_body(ent_hbm, rel_hbm, h_idx, r_idx, t_idx, out_hbm,
                hix, rix, tix, hbuf, rbuf, tbuf, qbuf, outv,
                hsem0, rsem0, tsem0, hsem1, rsem1, tsem1):
    sems = ((hsem0, rsem0, tsem0), (hsem1, rsem1, tsem1))
    nc = 2
    wid = lax.axis_index("s") * nc + lax.axis_index("c")
    base = wid * PER_W

    ih = pltpu.make_async_copy(h_idx.at[pl.ds(base, PER_W)], hix, hsem0)
    ir = pltpu.make_async_copy(r_idx.at[pl.ds(base, PER_W)], rix, rsem0)
    it = pltpu.make_async_copy(t_idx.at[pl.ds(base, PER_W)], tix, tsem0)
    ih.start(); ir.start(); it.start()
    ih.wait(); ir.wait(); it.wait()

    iota = lax.iota(jnp.int32, LANES)
    perms = {s: iota ^ s for s in (8, 4, 2, 1)}

    def copies(c, b):
        hs, rs, ts = sems[b]
        sl = pl.ds(c * CHUNK, CHUNK)
        return (pltpu.make_async_copy(ent_hbm.at[hix.at[sl]], hbuf.at[b], hs),
                pltpu.make_async_copy(rel_hbm.at[rix.at[sl]], rbuf.at[b], rs),
                pltpu.make_async_copy(ent_hbm.at[tix.at[sl]], tbuf.at[b], ts))

    def start(c, b):
        for cp in copies(c, b):
            cp.start()

    def wait(c, b):
        for cp in copies(c, b):
            cp.wait()

    def combine(a, bb, s):
        keep = (iota & s) == 0
        return jnp.where(keep, a + _shuffle(a, perms[s]),
                         bb + _shuffle(bb, perms[s]))

    def pair(c, b, i):
        g = lax.shift_right_logical(i, 3)
        ii = jnp.bitwise_and(i, 7)
        j0 = jnp.bitwise_or(
            jnp.bitwise_or(lax.shift_left(jnp.bitwise_and(ii, 1), 2),
                           jnp.bitwise_and(ii, 2)),
            jnp.bitwise_and(lax.shift_right_logical(ii, 2), 1))
        qbase = g * LANES + j0
        accs = []
        for m_off in (0, 8):
            row = qbase + m_off
            acc = jnp.zeros((LANES,), jnp.float32)
            for k in range(DIM // LANES):
                sl_r = pl.ds(k * LANES, LANES)
                sl_i = pl.ds(DIM + k * LANES, LANES)
                hr = hbuf[b, row, sl_r]; hi = hbuf[b, row, sl_i]
                tr = tbuf[b, row, sl_r]; ti = tbuf[b, row, sl_i]
                p = hr * tr + hi * ti
                q = hr * ti - hi * tr
                rr = rbuf[b, row, sl_r]; ri = rbuf[b, row, sl_i]
                acc = acc + (rr * p + ri * q)
            accs.append(acc)
        qbuf[i, :] = combine(accs[0], accs[1], 8)

    start(0, 0)

    def chunk_pair(cc, carry):
        for b in (0, 1):
            c = 2 * cc + b
            if b == 0:
                start(c + 1, 1)
            else:
                @pl.when(cc < NCHUNKS // 2 - 1)
                def _():
                    start(c + 1, 0)
            wait(c, b)

            def qstep(i, inner):
                pair(c, b, i)
                return inner
            lax.fori_loop(0, CHUNK // 2, qstep, jnp.int32(0))

            def gstep(g, inner):
                l2 = [combine(qbuf[8 * g + 2 * i, :],
                              qbuf[8 * g + 2 * i + 1, :], 4)
                      for i in range(4)]
                vec = combine(combine(l2[0], l2[1], 2),
                              combine(l2[2], l2[3], 2), 1)
                outv[pl.ds(c * CHUNK + g * LANES, LANES)] = vec
                return inner
            lax.fori_loop(0, CHUNK // LANES, gstep, jnp.int32(0))
        return carry

    lax.fori_loop(0, NCHUNKS // 2, chunk_pair, jnp.int32(0))

    pltpu.sync_copy(outv, out_hbm.at[pl.ds(base, PER_W)])


def kernel(ent, rel, h, r, t):
    mesh = plsc.VectorSubcoreMesh(core_axis_name="c", subcore_axis_name="s")
    run = functools.partial(
        pl.kernel,
        mesh=mesh,
        out_type=jax.ShapeDtypeStruct((BATCH,), jnp.float32),
        scratch_types=[
            pltpu.VMEM((PER_W,), jnp.int32),
            pltpu.VMEM((PER_W,), jnp.int32),
            pltpu.VMEM((PER_W,), jnp.int32),
            pltpu.VMEM((2, CHUNK, 2 * DIM), jnp.float32),
            pltpu.VMEM((2, CHUNK, 2 * DIM), jnp.float32),
            pltpu.VMEM((2, CHUNK, 2 * DIM), jnp.float32),
            pltpu.VMEM((CHUNK // 2, LANES), jnp.float32),
            pltpu.VMEM((PER_W,), jnp.float32),
            pltpu.SemaphoreType.DMA,
            pltpu.SemaphoreType.DMA,
            pltpu.SemaphoreType.DMA,
            pltpu.SemaphoreType.DMA,
            pltpu.SemaphoreType.DMA,
            pltpu.SemaphoreType.DMA,
        ],
    )(_score_body)
    return run(ent, rel, h, r, t)
